# Initial kernel scaffold; baseline (speedup 1.0000x reference)
import jax
import jax.numpy as jnp
from jax.experimental import pallas as pl


def kernel(x, edge_index, batch, Wl1, Wr1, att1, b1, Wl2, Wr2, att2, b2, Wro, bro):
    raise NotImplementedError("write your pallas kernel here")


# speedup vs baseline: 1.1753x; 1.1753x over previous
import functools
import jax
import jax.numpy as jnp
from jax import lax
from jax.experimental import pallas as pl
from jax.experimental.pallas import tpu as pltpu, tpu_sc as plsc

N = 10000
E = 160000
IN = 384
HID = 256
HEADS = 4
TOK = 4

NC, NS, L = 2, 16, 16
NW = NC * NS
E_REAL = E + N
EPAD = 170496
PER_TILE = EPAD // NW
NBATCH = PER_TILE // L
PER_TILE_SC = EPAD // NS
NBATCH_SC = PER_TILE_SC // L
ROWS_PER_TILE = 624
ROWS_TAIL = N - NS * ROWS_PER_TILE


def _rows_copy(mk_src, mk_dst, sid):
    r0 = sid * ROWS_PER_TILE
    pltpu.sync_copy(mk_src(r0, ROWS_PER_TILE), mk_dst(r0, ROWS_PER_TILE))
    @pl.when(sid == NS - 1)
    def _():
        t0 = NS * ROWS_PER_TILE
        pltpu.sync_copy(mk_src(t0, ROWS_TAIL), mk_dst(t0, ROWS_TAIL))

_mesh = lambda: plsc.VectorSubcoreMesh(core_axis_name="c", subcore_axis_name="s",
                                       num_cores=NC, num_subcores=NS)


def _mm(x, w, bn=512):
    m, k = x.shape
    n = w.shape[1]
    bm = 400
    def body(xr, wr, outr):
        outr[...] = jnp.dot(xr[...], wr[...], preferred_element_type=jnp.float32)
    return pl.pallas_call(
        body,
        grid=(m // bm, n // bn),
        in_specs=[pl.BlockSpec((bm, k), lambda i, j: (i, 0)),
                  pl.BlockSpec((k, bn), lambda i, j: (0, j))],
        out_specs=pl.BlockSpec((bm, bn), lambda i, j: (i, j)),
        out_shape=jax.ShapeDtypeStruct((m, n), jnp.float32),
    )(x, w)


def _edge_ex(xl, xr, att_flat, src, dst, zer16, heads):
    hf = heads * HID

    def body(xl_hbm, xr_hbm, att_hbm, src_hbm, dst_hbm, zer_hbm,
             ex_hbm, den_hbm,
             att_v, srci, dsti, xlr, xrr, exb, den_acc, sem1, sem2):
        cid = lax.axis_index("c")
        sid = lax.axis_index("s")
        wid = sid * NC + cid
        _rows_copy(lambda r, n: zer_hbm.at[pl.ds(r, n)],
                   lambda r, n: den_acc.at[pl.ds(r, n)], sid)
        pltpu.sync_copy(att_hbm, att_v)
        plsc.subcore_barrier()
        lane = lax.iota(jnp.int32, L)
        base_t = wid * PER_TILE

        def zrow(i, carry):
            exb[i] = jnp.zeros((L,), jnp.float32)
            return carry
        lax.fori_loop(0, L, zrow, 0)

        def batch(b, carry):
            base = base_t + b * L
            pltpu.sync_copy(src_hbm.at[pl.ds(base, L)], srci)
            pltpu.sync_copy(dst_hbm.at[pl.ds(base, L)], dsti)
            cp1 = pltpu.async_copy(xl_hbm.at[srci], xlr, sem1)
            cp2 = pltpu.async_copy(xr_hbm.at[dsti], xrr, sem2)
            cp1.wait()
            cp2.wait()
            valid = (base + lane) < E_REAL
            for h in range(heads):
                def floop(f, acc):
                    col = jnp.full((L,), h * HID + f, jnp.int32)
                    a = plsc.load_gather(xlr, [lane, col])
                    bb = plsc.load_gather(xrr, [lane, col])
                    av = plsc.load_gather(att_v, [col])
                    t = a + bb
                    m = jnp.maximum(t, 0.2 * t)
                    return acc + m * av
                logits = lax.fori_loop(0, HID, floop,
                                       jnp.zeros((L,), jnp.float32),
                                       unroll=4)
                exv = jnp.where(valid, jnp.exp(logits), 0.0)
                plsc.store_scatter(exb, [lane, jnp.full((L,), h, jnp.int32)],
                                   exv)
            pltpu.sync_copy(exb, den_acc.at[dsti], add=True)
            pltpu.sync_copy(exb, ex_hbm.at[pl.ds(base, L)])
            return carry
        lax.fori_loop(0, NBATCH, batch, 0)
        plsc.subcore_barrier()
        _rows_copy(lambda r, n: den_acc.at[pl.ds(r, n)],
                   lambda r, n: den_hbm.at[cid, pl.ds(r, n)], sid)

    fn = pl.kernel(
        body,
        out_type=(jax.ShapeDtypeStruct((EPAD, 16), jnp.float32),
                  jax.ShapeDtypeStruct((2, N, 16), jnp.float32)),
        mesh=_mesh(),
        compiler_params=pltpu.CompilerParams(use_tc_tiling_on_sc=False, needs_layout_passes=False),
        scratch_types=[
            pltpu.VMEM((hf,), jnp.float32),
            pltpu.VMEM((L,), jnp.int32),
            pltpu.VMEM((L,), jnp.int32),
            pltpu.VMEM((L, hf), jnp.float32),
            pltpu.VMEM((L, hf), jnp.float32),
            pltpu.VMEM((L, 16), jnp.float32),
            pltpu.VMEM_SHARED((N, 16), jnp.float32),
            pltpu.SemaphoreType.DMA,
            pltpu.SemaphoreType.DMA,
        ],
    )
    return fn(xl, xr, att_flat, src, dst, zer16)


def _edge_aggr(xlt, ex, src, dst, zer128, nch):
    def body(xlt_hbm, ex_hbm, src_hbm, dst_hbm, zer_hbm,
             out_hbm,
             srci, dsti, rows, exb, acc_sh, sem1):
        cid = lax.axis_index("c")
        sid = lax.axis_index("s")
        base_t = sid * PER_TILE_SC
        for k in range(nch // 2):
            c = 2 * k + cid
            _rows_copy(lambda r, n: zer_hbm.at[pl.ds(r, n)],
                       lambda r, n: acc_sh.at[pl.ds(r, n)], sid)
            plsc.subcore_barrier()

            lane = lax.iota(jnp.int32, L)

            def batch(b, carry):
                base = base_t + b * L
                pltpu.sync_copy(src_hbm.at[pl.ds(base, L)], srci)
                srci[:] = srci[:] + c * N
                pltpu.sync_copy(dst_hbm.at[pl.ds(base, L)], dsti)
                pltpu.async_copy(xlt_hbm.at[srci], rows, sem1).wait()
                pltpu.sync_copy(ex_hbm.at[pl.ds(base, L)], exb)
                exv = plsc.load_gather(exb, [lane, jnp.full((L,), k, jnp.int32)])

                def floop(f, carry2):
                    col = jnp.full((L,), f, jnp.int32)
                    v = plsc.load_gather(rows, [lane, col])
                    plsc.store_scatter(rows, [lane, col], v * exv)
                    return carry2
                lax.fori_loop(0, 128, floop, 0, unroll=4)
                pltpu.sync_copy(rows, acc_sh.at[dsti], add=True)
                return carry
            lax.fori_loop(0, NBATCH_SC, batch, 0)
            plsc.subcore_barrier()
            _rows_copy(lambda r, n: acc_sh.at[pl.ds(r, n)],
                       lambda r, n: out_hbm.at[c, pl.ds(r, n)], sid)
            plsc.subcore_barrier()

    fn = pl.kernel(
        body,
        out_type=jax.ShapeDtypeStruct((nch, N, 128), jnp.float32),
        mesh=_mesh(),
        compiler_params=pltpu.CompilerParams(use_tc_tiling_on_sc=False, needs_layout_passes=False),
        scratch_types=[
            pltpu.VMEM((L,), jnp.int32),
            pltpu.VMEM((L,), jnp.int32),
            pltpu.VMEM((L, 128), jnp.float32),
            pltpu.VMEM((L, 16), jnp.float32),
            pltpu.VMEM_SHARED((N, 128), jnp.float32),
            pltpu.SemaphoreType.DMA,
        ],
    )
    return fn(xlt, ex, src, dst, zer128)


def _norm(chunks, den, bias, heads, relu):
    hf = heads * HID
    bm = 400
    def body(xr, dr, br, outr):
        x = xr[...]
        d = dr[...][0] + dr[...][1]
        b = br[...]
        cols = []
        for h in range(heads):
            seg = x[:, h * HID:(h + 1) * HID]
            dh = d[:, h:h + 1] + 1e-16
            cols.append(seg / dh)
        o = jnp.concatenate(cols, axis=1) + b
        if relu:
            o = jnp.maximum(o, 0.0)
        outr[...] = o
    return pl.pallas_call(
        body,
        grid=(N // bm,),
        in_specs=[pl.BlockSpec((bm, hf), lambda i: (i, 0)),
                  pl.BlockSpec((2, bm, 16), lambda i: (0, i, 0)),
                  pl.BlockSpec((1, hf), lambda i: (0, 0))],
        out_specs=pl.BlockSpec((bm, hf), lambda i: (i, 0)),
        out_shape=jax.ShapeDtypeStruct((N, hf), jnp.float32),
    )(chunks, den, bias.reshape(1, hf))


def _pool_readout(h, wro, bro):
    bm = 400
    nsteps = N // bm
    def body(hr, wr, br, outr, acc):
        i = pl.program_id(0)
        @pl.when(i == 0)
        def _():
            acc[...] = jnp.zeros_like(acc)
        acc[0:1, :] += jnp.sum(hr[...], axis=0, keepdims=True)
        @pl.when(i == nsteps - 1)
        def _():
            pooled = acc[0:1, :] / float(N)
            outr[...] = jnp.dot(pooled, wr[...],
                                preferred_element_type=jnp.float32) + br[...]
    return pl.pallas_call(
        body,
        grid=(nsteps,),
        in_specs=[pl.BlockSpec((bm, HID), lambda i: (i, 0)),
                  pl.BlockSpec((HID, TOK * IN), lambda i: (0, 0)),
                  pl.BlockSpec((1, TOK * IN), lambda i: (0, 0))],
        out_specs=pl.BlockSpec((1, TOK * IN), lambda i: (0, 0)),
        out_shape=jax.ShapeDtypeStruct((1, TOK * IN), jnp.float32),
        scratch_shapes=[pltpu.VMEM((8, HID), jnp.float32)],
    )(h, wro, bro.reshape(1, TOK * IN))


def kernel(x, edge_index, batch, Wl1, Wr1, att1, b1, Wl2, Wr2, att2, b2, Wro, bro):
    loops = jnp.arange(N, dtype=jnp.int32)
    padn = EPAD - E_REAL
    src = jnp.concatenate([edge_index[0].astype(jnp.int32), loops,
                           jnp.zeros((padn,), jnp.int32)])
    dst = jnp.concatenate([edge_index[1].astype(jnp.int32), loops,
                           jnp.zeros((padn,), jnp.int32)])
    zer16 = jnp.zeros((N, 16), jnp.float32)
    zer128 = jnp.zeros((N, 128), jnp.float32)

    xcat = _mm(x, jnp.concatenate([Wl1, Wr1], axis=1))
    xl1, xr1 = xcat[:, :HEADS * HID], xcat[:, HEADS * HID:]
    ex1, den1 = _edge_ex(xl1, xr1, att1.reshape(-1), src, dst, zer16, HEADS)
    xlt1 = xl1.reshape(N, 8, 128).transpose(1, 0, 2).reshape(8 * N, 128)
    ch1 = _edge_aggr(xlt1, ex1, src, dst, zer128, 8)
    h1full = ch1.transpose(1, 0, 2).reshape(N, HEADS * HID)
    h1 = _norm(h1full, den1, b1, HEADS, relu=True)

    xcat2 = _mm(h1, jnp.concatenate([Wl2, Wr2], axis=1), bn=256)
    xl2, xr2 = xcat2[:, :HID], xcat2[:, HID:]
    ex2, den2 = _edge_ex(xl2, xr2, att2.reshape(-1), src, dst, zer16, 1)
    xlt2 = xl2.reshape(N, 2, 128).transpose(1, 0, 2).reshape(2 * N, 128)
    ch2 = _edge_aggr(xlt2, ex2, src, dst, zer128, 2)
    h2full = ch2.transpose(1, 0, 2).reshape(N, HID)
    h2 = _norm(h2full, den2, b2, 1, relu=False)

    out = _pool_readout(h2, Wro, bro)
    return out.reshape(1, TOK, IN)


# speedup vs baseline: 1.5270x; 1.2993x over previous
import functools
import jax
import jax.numpy as jnp
from jax import lax
from jax.experimental import pallas as pl
from jax.experimental.pallas import tpu as pltpu, tpu_sc as plsc

N = 10000
E = 160000
IN = 384
HID = 256
HEADS = 4
TOK = 4

NC, NS, L = 2, 16, 16
NW = NC * NS
E_REAL = E + N
EPAD = 172032
PER_TILE = EPAD // NW
NBATCH = PER_TILE // L
PER_TILE_SC = EPAD // NS
BB = 64
NBATCH_SC = PER_TILE_SC // BB
NPAIR = NBATCH_SC // 2
ROWS_PER_TILE = 624
ROWS_TAIL = N - NS * ROWS_PER_TILE


def _rows_copy(mk_src, mk_dst, sid):
    r0 = sid * ROWS_PER_TILE
    pltpu.sync_copy(mk_src(r0, ROWS_PER_TILE), mk_dst(r0, ROWS_PER_TILE))
    @pl.when(sid == NS - 1)
    def _():
        t0 = NS * ROWS_PER_TILE
        pltpu.sync_copy(mk_src(t0, ROWS_TAIL), mk_dst(t0, ROWS_TAIL))

_mesh = lambda: plsc.VectorSubcoreMesh(core_axis_name="c", subcore_axis_name="s",
                                       num_cores=NC, num_subcores=NS)


def _mm(x, w, bn=512):
    m, k = x.shape
    n = w.shape[1]
    bm = 400
    def body(xr, wr, outr):
        outr[...] = jnp.dot(xr[...], wr[...], preferred_element_type=jnp.float32)
    return pl.pallas_call(
        body,
        grid=(m // bm, n // bn),
        in_specs=[pl.BlockSpec((bm, k), lambda i, j: (i, 0)),
                  pl.BlockSpec((k, bn), lambda i, j: (0, j))],
        out_specs=pl.BlockSpec((bm, bn), lambda i, j: (i, j)),
        out_shape=jax.ShapeDtypeStruct((m, n), jnp.float32),
    )(x, w)


def _edge_ex(xl, xr, att_flat, src, dst, zer16, heads):
    hf = heads * HID

    def body(xl_hbm, xr_hbm, att_hbm, src_hbm, dst_hbm, zer_hbm,
             ex_hbm, den_hbm,
             att_v, srci, dsti, xlr, xrr, exb, den_acc, sem1, sem2):
        cid = lax.axis_index("c")
        sid = lax.axis_index("s")
        wid = sid * NC + cid
        _rows_copy(lambda r, n: zer_hbm.at[pl.ds(r, n)],
                   lambda r, n: den_acc.at[pl.ds(r, n)], sid)
        pltpu.sync_copy(att_hbm, att_v)
        plsc.subcore_barrier()
        lane = lax.iota(jnp.int32, L)
        base_t = wid * PER_TILE

        def zrow(i, carry):
            exb[i] = jnp.zeros((L,), jnp.float32)
            return carry
        lax.fori_loop(0, L, zrow, 0)

        def batch(b, carry):
            base = base_t + b * L
            pltpu.sync_copy(src_hbm.at[pl.ds(base, L)], srci)
            pltpu.sync_copy(dst_hbm.at[pl.ds(base, L)], dsti)
            cp1 = pltpu.async_copy(xl_hbm.at[srci], xlr, sem1)
            cp2 = pltpu.async_copy(xr_hbm.at[dsti], xrr, sem2)
            cp1.wait()
            cp2.wait()
            valid = (base + lane) < E_REAL
            for h in range(heads):
                def floop(f, acc):
                    col = jnp.full((L,), h * HID + f, jnp.int32)
                    a = plsc.load_gather(xlr, [lane, col])
                    bb = plsc.load_gather(xrr, [lane, col])
                    av = plsc.load_gather(att_v, [col])
                    t = a + bb
                    m = jnp.maximum(t, 0.2 * t)
                    return acc + m * av
                logits = lax.fori_loop(0, HID, floop,
                                       jnp.zeros((L,), jnp.float32),
                                       unroll=4)
                exv = jnp.where(valid, jnp.exp(logits), 0.0)
                plsc.store_scatter(exb, [lane, jnp.full((L,), h, jnp.int32)],
                                   exv)
            pltpu.sync_copy(exb, den_acc.at[dsti], add=True)
            pltpu.sync_copy(exb, ex_hbm.at[pl.ds(base, L)])
            return carry
        lax.fori_loop(0, NBATCH, batch, 0)
        plsc.subcore_barrier()
        _rows_copy(lambda r, n: den_acc.at[pl.ds(r, n)],
                   lambda r, n: den_hbm.at[cid, pl.ds(r, n)], sid)

    fn = pl.kernel(
        body,
        out_type=(jax.ShapeDtypeStruct((EPAD, 16), jnp.float32),
                  jax.ShapeDtypeStruct((2, N, 16), jnp.float32)),
        mesh=_mesh(),
        compiler_params=pltpu.CompilerParams(use_tc_tiling_on_sc=False, needs_layout_passes=False),
        scratch_types=[
            pltpu.VMEM((hf,), jnp.float32),
            pltpu.VMEM((L,), jnp.int32),
            pltpu.VMEM((L,), jnp.int32),
            pltpu.VMEM((L, hf), jnp.float32),
            pltpu.VMEM((L, hf), jnp.float32),
            pltpu.VMEM((L, 16), jnp.float32),
            pltpu.VMEM_SHARED((N, 16), jnp.float32),
            pltpu.SemaphoreType.DMA,
            pltpu.SemaphoreType.DMA,
        ],
    )
    return fn(xl, xr, att_flat, src, dst, zer16)


def _edge_aggr(xlt, ex, src, dst, zer128, nch):
    def body(xlt_hbm, ex_hbm, src_hbm, dst_hbm, zer_hbm,
             out_hbm,
             srcc, dsti, rows0, rows1, exb, acc_sh, semg0, semg1):
        cid = lax.axis_index("c")
        sid = lax.axis_index("s")
        base_t = sid * PER_TILE_SC
        lane = lax.iota(jnp.int32, L)

        pltpu.sync_copy(src_hbm.at[pl.ds(base_t, PER_TILE_SC)], srcc)

        def shift(d):
            def sloop(j, carry):
                sl = pl.ds(j * L, L)
                srcc[sl] = srcc[sl] + d
                return carry
            lax.fori_loop(0, PER_TILE_SC // L, sloop, 0, unroll=8)

        shift(cid * N)
        for k in range(nch // 2):
            if k > 0:
                shift(2 * N)
            c = 2 * k + cid
            _rows_copy(lambda r, n: zer_hbm.at[pl.ds(r, n)],
                       lambda r, n: acc_sh.at[pl.ds(r, n)], sid)
            plsc.subcore_barrier()

            def scale_scatter(b, rows):
                pltpu.sync_copy(ex_hbm.at[pl.ds(base_t + b * BB, BB)], exb)
                exvs = [plsc.load_gather(
                    exb, [lane + g * L, jnp.full((L,), k, jnp.int32)])
                    for g in range(BB // L)]

                def floop(f, carry2):
                    col = jnp.full((L,), f, jnp.int32)
                    for g in range(BB // L):
                        rl = lane + g * L
                        v = plsc.load_gather(rows, [rl, col])
                        plsc.store_scatter(rows, [rl, col], v * exvs[g])
                    return carry2
                lax.fori_loop(0, 128, floop, 0, unroll=2)
                pltpu.sync_copy(dst_hbm.at[pl.ds(base_t + b * BB, BB)], dsti)
                pltpu.sync_copy(rows, acc_sh.at[dsti], add=True)

            def gather(b, rows, sem):
                return pltpu.async_copy(
                    xlt_hbm.at[srcc.at[pl.ds(b * BB, BB)]], rows, sem)

            gather(0, rows0, semg0)

            def pair(p, carry):
                b0 = 2 * p
                pltpu.make_async_copy(
                    xlt_hbm.at[srcc.at[pl.ds(b0 * BB, BB)]], rows0,
                    semg0).wait()
                gather(b0 + 1, rows1, semg1)
                scale_scatter(b0, rows0)
                @pl.when(p < NPAIR - 1)
                def _():
                    gather(b0 + 2, rows0, semg0)
                pltpu.make_async_copy(
                    xlt_hbm.at[srcc.at[pl.ds((b0 + 1) * BB, BB)]], rows1,
                    semg1).wait()
                scale_scatter(b0 + 1, rows1)
                return carry
            lax.fori_loop(0, NPAIR, pair, 0)
            plsc.subcore_barrier()
            _rows_copy(lambda r, n: acc_sh.at[pl.ds(r, n)],
                       lambda r, n: out_hbm.at[c, pl.ds(r, n)], sid)
            plsc.subcore_barrier()

    fn = pl.kernel(
        body,
        out_type=jax.ShapeDtypeStruct((nch, N, 128), jnp.float32),
        mesh=_mesh(),
        compiler_params=pltpu.CompilerParams(use_tc_tiling_on_sc=False, needs_layout_passes=False),
        scratch_types=[
            pltpu.VMEM((PER_TILE_SC,), jnp.int32),
            pltpu.VMEM((BB,), jnp.int32),
            pltpu.VMEM((BB, 128), jnp.float32),
            pltpu.VMEM((BB, 128), jnp.float32),
            pltpu.VMEM((BB, 16), jnp.float32),
            pltpu.VMEM_SHARED((N, 128), jnp.float32),
            pltpu.SemaphoreType.DMA,
            pltpu.SemaphoreType.DMA,
        ],
    )
    return fn(xlt, ex, src, dst, zer128)


def _norm(chunks, den, bias, heads, relu):
    hf = heads * HID
    bm = 400
    def body(xr, dr, br, outr):
        x = xr[...]
        d = dr[...][0] + dr[...][1]
        b = br[...]
        cols = []
        for h in range(heads):
            seg = x[:, h * HID:(h + 1) * HID]
            dh = d[:, h:h + 1] + 1e-16
            cols.append(seg / dh)
        o = jnp.concatenate(cols, axis=1) + b
        if relu:
            o = jnp.maximum(o, 0.0)
        outr[...] = o
    return pl.pallas_call(
        body,
        grid=(N // bm,),
        in_specs=[pl.BlockSpec((bm, hf), lambda i: (i, 0)),
                  pl.BlockSpec((2, bm, 16), lambda i: (0, i, 0)),
                  pl.BlockSpec((1, hf), lambda i: (0, 0))],
        out_specs=pl.BlockSpec((bm, hf), lambda i: (i, 0)),
        out_shape=jax.ShapeDtypeStruct((N, hf), jnp.float32),
    )(chunks, den, bias.reshape(1, hf))


def _pool_readout(h, wro, bro):
    bm = 400
    nsteps = N // bm
    def body(hr, wr, br, outr, acc):
        i = pl.program_id(0)
        @pl.when(i == 0)
        def _():
            acc[...] = jnp.zeros_like(acc)
        acc[0:1, :] += jnp.sum(hr[...], axis=0, keepdims=True)
        @pl.when(i == nsteps - 1)
        def _():
            pooled = acc[0:1, :] / float(N)
            outr[...] = jnp.dot(pooled, wr[...],
                                preferred_element_type=jnp.float32) + br[...]
    return pl.pallas_call(
        body,
        grid=(nsteps,),
        in_specs=[pl.BlockSpec((bm, HID), lambda i: (i, 0)),
                  pl.BlockSpec((HID, TOK * IN), lambda i: (0, 0)),
                  pl.BlockSpec((1, TOK * IN), lambda i: (0, 0))],
        out_specs=pl.BlockSpec((1, TOK * IN), lambda i: (0, 0)),
        out_shape=jax.ShapeDtypeStruct((1, TOK * IN), jnp.float32),
        scratch_shapes=[pltpu.VMEM((8, HID), jnp.float32)],
    )(h, wro, bro.reshape(1, TOK * IN))


def kernel(x, edge_index, batch, Wl1, Wr1, att1, b1, Wl2, Wr2, att2, b2, Wro, bro):
    loops = jnp.arange(N, dtype=jnp.int32)
    padn = EPAD - E_REAL
    src = jnp.concatenate([edge_index[0].astype(jnp.int32), loops,
                           jnp.zeros((padn,), jnp.int32)])
    dst = jnp.concatenate([edge_index[1].astype(jnp.int32), loops,
                           jnp.zeros((padn,), jnp.int32)])
    zer16 = jnp.zeros((N, 16), jnp.float32)
    zer128 = jnp.zeros((N, 128), jnp.float32)

    xcat = _mm(x, jnp.concatenate([Wl1, Wr1], axis=1))
    xl1, xr1 = xcat[:, :HEADS * HID], xcat[:, HEADS * HID:]
    ex1, den1 = _edge_ex(xl1, xr1, att1.reshape(-1), src, dst, zer16, HEADS)
    xlt1 = xl1.reshape(N, 8, 128).transpose(1, 0, 2).reshape(8 * N, 128)
    ch1 = _edge_aggr(xlt1, ex1, src, dst, zer128, 8)
    h1full = ch1.transpose(1, 0, 2).reshape(N, HEADS * HID)
    h1 = _norm(h1full, den1, b1, HEADS, relu=True)

    xcat2 = _mm(h1, jnp.concatenate([Wl2, Wr2], axis=1), bn=256)
    xl2, xr2 = xcat2[:, :HID], xcat2[:, HID:]
    ex2, den2 = _edge_ex(xl2, xr2, att2.reshape(-1), src, dst, zer16, 1)
    xlt2 = xl2.reshape(N, 2, 128).transpose(1, 0, 2).reshape(2 * N, 128)
    ch2 = _edge_aggr(xlt2, ex2, src, dst, zer128, 2)
    h2full = ch2.transpose(1, 0, 2).reshape(N, HID)
    h2 = _norm(h2full, den2, b2, 1, relu=False)

    out = _pool_readout(h2, Wro, bro)
    return out.reshape(1, TOK, IN)


# speedup vs baseline: 1.6759x; 1.0975x over previous
import functools
import jax
import jax.numpy as jnp
from jax import lax
from jax.experimental import pallas as pl
from jax.experimental.pallas import tpu as pltpu, tpu_sc as plsc

N = 10000
E = 160000
IN = 384
HID = 256
HEADS = 4
TOK = 4

NC, NS, L = 2, 16, 16
NW = NC * NS
E_REAL = E + N
EPAD = 172032
PER_TILE = EPAD // NW
NBATCH = PER_TILE // L
PER_TILE_SC = EPAD // NS
BB = 64
NBATCH_SC = PER_TILE_SC // BB
NPAIR = NBATCH_SC // 2
ROWS_PER_TILE = 624
ROWS_TAIL = N - NS * ROWS_PER_TILE


def _rows_copy(mk_src, mk_dst, sid):
    r0 = sid * ROWS_PER_TILE
    pltpu.sync_copy(mk_src(r0, ROWS_PER_TILE), mk_dst(r0, ROWS_PER_TILE))
    @pl.when(sid == NS - 1)
    def _():
        t0 = NS * ROWS_PER_TILE
        pltpu.sync_copy(mk_src(t0, ROWS_TAIL), mk_dst(t0, ROWS_TAIL))

_mesh = lambda: plsc.VectorSubcoreMesh(core_axis_name="c", subcore_axis_name="s",
                                       num_cores=NC, num_subcores=NS)


def _mm(x, w, bn=512):
    m, k = x.shape
    n = w.shape[1]
    bm = 400
    def body(xr, wr, outr):
        outr[...] = jnp.dot(xr[...], wr[...], preferred_element_type=jnp.float32)
    return pl.pallas_call(
        body,
        grid=(m // bm, n // bn),
        in_specs=[pl.BlockSpec((bm, k), lambda i, j: (i, 0)),
                  pl.BlockSpec((k, bn), lambda i, j: (0, j))],
        out_specs=pl.BlockSpec((bm, bn), lambda i, j: (i, j)),
        out_shape=jax.ShapeDtypeStruct((m, n), jnp.float32),
    )(x, w)


def _edge_ex(xl, xr, att_flat, src, dst, zer16, heads):
    hf = heads * HID

    def body(xl_hbm, xr_hbm, att_hbm, src_hbm, dst_hbm, zer_hbm,
             ex_hbm, den_hbm,
             att_v, srcv, dstv, dsti, xlr0, xrr0, xlr1, xrr1, exb, den_acc,
             sa0, sb0, sa1, sb1):
        cid = lax.axis_index("c")
        sid = lax.axis_index("s")
        wid = sid * NC + cid
        _rows_copy(lambda r, n: zer_hbm.at[pl.ds(r, n)],
                   lambda r, n: den_acc.at[pl.ds(r, n)], sid)
        pltpu.sync_copy(att_hbm, att_v)
        lane = lax.iota(jnp.int32, L)
        base_t = wid * PER_TILE
        pltpu.sync_copy(src_hbm.at[pl.ds(base_t, PER_TILE)], srcv)
        pltpu.sync_copy(dst_hbm.at[pl.ds(base_t, PER_TILE)], dstv)
        plsc.subcore_barrier()

        def zrow(i, carry):
            exb[i] = jnp.zeros((L,), jnp.float32)
            return carry
        lax.fori_loop(0, L, zrow, 0)

        def gathers(b, xlr, xrr, sa, sb):
            pltpu.async_copy(xl_hbm.at[srcv.at[pl.ds(b * L, L)]], xlr, sa)
            pltpu.async_copy(xr_hbm.at[dstv.at[pl.ds(b * L, L)]], xrr, sb)

        def gwait(b, xlr, xrr, sa, sb):
            pltpu.make_async_copy(
                xl_hbm.at[srcv.at[pl.ds(b * L, L)]], xlr, sa).wait()
            pltpu.make_async_copy(
                xr_hbm.at[dstv.at[pl.ds(b * L, L)]], xrr, sb).wait()

        def compute(b, xlr, xrr):
            base = base_t + b * L
            valid = (base + lane) < E_REAL
            for h in range(heads):
                def floop(f, accs):
                    a0, a1, a2, a3 = accs
                    outs = []
                    for j, acc in enumerate((a0, a1, a2, a3)):
                        col = jnp.full((L,), h * HID + f * 4 + j, jnp.int32)
                        a = plsc.load_gather(xlr, [lane, col])
                        bb = plsc.load_gather(xrr, [lane, col])
                        av = plsc.load_gather(att_v, [col])
                        t = a + bb
                        m = jnp.maximum(t, 0.2 * t)
                        outs.append(acc + m * av)
                    return tuple(outs)
                z = jnp.zeros((L,), jnp.float32)
                a0, a1, a2, a3 = lax.fori_loop(0, HID // 4, floop,
                                               (z, z, z, z), unroll=2)
                logits = (a0 + a1) + (a2 + a3)
                exv = jnp.where(valid, jnp.exp(logits), 0.0)
                plsc.store_scatter(exb, [lane, jnp.full((L,), h, jnp.int32)],
                                   exv)
            pltpu.sync_copy(dst_hbm.at[pl.ds(base, L)], dsti)
            pltpu.sync_copy(exb, den_acc.at[dsti], add=True)
            pltpu.sync_copy(exb, ex_hbm.at[pl.ds(base, L)])

        gathers(0, xlr0, xrr0, sa0, sb0)

        def pair(p, carry):
            b0 = 2 * p
            gwait(b0, xlr0, xrr0, sa0, sb0)
            gathers(b0 + 1, xlr1, xrr1, sa1, sb1)
            compute(b0, xlr0, xrr0)
            @pl.when(p < NBATCH // 2 - 1)
            def _():
                gathers(b0 + 2, xlr0, xrr0, sa0, sb0)
            gwait(b0 + 1, xlr1, xrr1, sa1, sb1)
            compute(b0 + 1, xlr1, xrr1)
            return carry
        lax.fori_loop(0, NBATCH // 2, pair, 0)
        plsc.subcore_barrier()
        _rows_copy(lambda r, n: den_acc.at[pl.ds(r, n)],
                   lambda r, n: den_hbm.at[cid, pl.ds(r, n)], sid)

    fn = pl.kernel(
        body,
        out_type=(jax.ShapeDtypeStruct((EPAD, 16), jnp.float32),
                  jax.ShapeDtypeStruct((2, N, 16), jnp.float32)),
        mesh=_mesh(),
        compiler_params=pltpu.CompilerParams(use_tc_tiling_on_sc=False, needs_layout_passes=False),
        scratch_types=[
            pltpu.VMEM((hf,), jnp.float32),
            pltpu.VMEM((PER_TILE,), jnp.int32),
            pltpu.VMEM((PER_TILE,), jnp.int32),
            pltpu.VMEM((L,), jnp.int32),
            pltpu.VMEM((L, hf), jnp.float32),
            pltpu.VMEM((L, hf), jnp.float32),
            pltpu.VMEM((L, hf), jnp.float32),
            pltpu.VMEM((L, hf), jnp.float32),
            pltpu.VMEM((L, 16), jnp.float32),
            pltpu.VMEM_SHARED((N, 16), jnp.float32),
            pltpu.SemaphoreType.DMA,
            pltpu.SemaphoreType.DMA,
            pltpu.SemaphoreType.DMA,
            pltpu.SemaphoreType.DMA,
        ],
    )
    return fn(xl, xr, att_flat, src, dst, zer16)


def _edge_aggr(xlt, ex, src, dst, zer128, nch):
    def body(xlt_hbm, ex_hbm, src_hbm, dst_hbm, zer_hbm,
             out_hbm,
             srcc, dsti, rows0, rows1, exb, acc_sh, semg0, semg1):
        cid = lax.axis_index("c")
        sid = lax.axis_index("s")
        base_t = sid * PER_TILE_SC
        lane = lax.iota(jnp.int32, L)

        pltpu.sync_copy(src_hbm.at[pl.ds(base_t, PER_TILE_SC)], srcc)

        def shift(d):
            def sloop(j, carry):
                sl = pl.ds(j * L, L)
                srcc[sl] = srcc[sl] + d
                return carry
            lax.fori_loop(0, PER_TILE_SC // L, sloop, 0, unroll=8)

        shift(cid * N)
        for k in range(nch // 2):
            if k > 0:
                shift(2 * N)
            c = 2 * k + cid
            _rows_copy(lambda r, n: zer_hbm.at[pl.ds(r, n)],
                       lambda r, n: acc_sh.at[pl.ds(r, n)], sid)
            plsc.subcore_barrier()

            def scale_scatter(b, rows):
                pltpu.sync_copy(ex_hbm.at[pl.ds(base_t + b * BB, BB)], exb)
                exvs = [plsc.load_gather(
                    exb, [lane + g * L, jnp.full((L,), k, jnp.int32)])
                    for g in range(BB // L)]

                def floop(f, carry2):
                    col = jnp.full((L,), f, jnp.int32)
                    for g in range(BB // L):
                        rl = lane + g * L
                        v = plsc.load_gather(rows, [rl, col])
                        plsc.store_scatter(rows, [rl, col], v * exvs[g])
                    return carry2
                lax.fori_loop(0, 128, floop, 0, unroll=2)
                pltpu.sync_copy(dst_hbm.at[pl.ds(base_t + b * BB, BB)], dsti)
                pltpu.sync_copy(rows, acc_sh.at[dsti], add=True)

            def gather(b, rows, sem):
                return pltpu.async_copy(
                    xlt_hbm.at[srcc.at[pl.ds(b * BB, BB)]], rows, sem)

            gather(0, rows0, semg0)

            def pair(p, carry):
                b0 = 2 * p
                pltpu.make_async_copy(
                    xlt_hbm.at[srcc.at[pl.ds(b0 * BB, BB)]], rows0,
                    semg0).wait()
                gather(b0 + 1, rows1, semg1)
                scale_scatter(b0, rows0)
                @pl.when(p < NPAIR - 1)
                def _():
                    gather(b0 + 2, rows0, semg0)
                pltpu.make_async_copy(
                    xlt_hbm.at[srcc.at[pl.ds((b0 + 1) * BB, BB)]], rows1,
                    semg1).wait()
                scale_scatter(b0 + 1, rows1)
                return carry
            lax.fori_loop(0, NPAIR, pair, 0)
            plsc.subcore_barrier()
            _rows_copy(lambda r, n: acc_sh.at[pl.ds(r, n)],
                       lambda r, n: out_hbm.at[c, pl.ds(r, n)], sid)
            plsc.subcore_barrier()

    fn = pl.kernel(
        body,
        out_type=jax.ShapeDtypeStruct((nch, N, 128), jnp.float32),
        mesh=_mesh(),
        compiler_params=pltpu.CompilerParams(use_tc_tiling_on_sc=False, needs_layout_passes=False),
        scratch_types=[
            pltpu.VMEM((PER_TILE_SC,), jnp.int32),
            pltpu.VMEM((BB,), jnp.int32),
            pltpu.VMEM((BB, 128), jnp.float32),
            pltpu.VMEM((BB, 128), jnp.float32),
            pltpu.VMEM((BB, 16), jnp.float32),
            pltpu.VMEM_SHARED((N, 128), jnp.float32),
            pltpu.SemaphoreType.DMA,
            pltpu.SemaphoreType.DMA,
        ],
    )
    return fn(xlt, ex, src, dst, zer128)


def _norm(chunks, den, bias, heads, relu):
    hf = heads * HID
    bm = 400
    def body(xr, dr, br, outr):
        x = xr[...]
        d = dr[...][0] + dr[...][1]
        b = br[...]
        cols = []
        for h in range(heads):
            seg = x[:, h * HID:(h + 1) * HID]
            dh = d[:, h:h + 1] + 1e-16
            cols.append(seg / dh)
        o = jnp.concatenate(cols, axis=1) + b
        if relu:
            o = jnp.maximum(o, 0.0)
        outr[...] = o
    return pl.pallas_call(
        body,
        grid=(N // bm,),
        in_specs=[pl.BlockSpec((bm, hf), lambda i: (i, 0)),
                  pl.BlockSpec((2, bm, 16), lambda i: (0, i, 0)),
                  pl.BlockSpec((1, hf), lambda i: (0, 0))],
        out_specs=pl.BlockSpec((bm, hf), lambda i: (i, 0)),
        out_shape=jax.ShapeDtypeStruct((N, hf), jnp.float32),
    )(chunks, den, bias.reshape(1, hf))


def _pool_readout(h, wro, bro):
    bm = 400
    nsteps = N // bm
    def body(hr, wr, br, outr, acc):
        i = pl.program_id(0)
        @pl.when(i == 0)
        def _():
            acc[...] = jnp.zeros_like(acc)
        acc[0:1, :] += jnp.sum(hr[...], axis=0, keepdims=True)
        @pl.when(i == nsteps - 1)
        def _():
            pooled = acc[0:1, :] / float(N)
            outr[...] = jnp.dot(pooled, wr[...],
                                preferred_element_type=jnp.float32) + br[...]
    return pl.pallas_call(
        body,
        grid=(nsteps,),
        in_specs=[pl.BlockSpec((bm, HID), lambda i: (i, 0)),
                  pl.BlockSpec((HID, TOK * IN), lambda i: (0, 0)),
                  pl.BlockSpec((1, TOK * IN), lambda i: (0, 0))],
        out_specs=pl.BlockSpec((1, TOK * IN), lambda i: (0, 0)),
        out_shape=jax.ShapeDtypeStruct((1, TOK * IN), jnp.float32),
        scratch_shapes=[pltpu.VMEM((8, HID), jnp.float32)],
    )(h, wro, bro.reshape(1, TOK * IN))


def kernel(x, edge_index, batch, Wl1, Wr1, att1, b1, Wl2, Wr2, att2, b2, Wro, bro):
    loops = jnp.arange(N, dtype=jnp.int32)
    padn = EPAD - E_REAL
    src = jnp.concatenate([edge_index[0].astype(jnp.int32), loops,
                           jnp.zeros((padn,), jnp.int32)])
    dst = jnp.concatenate([edge_index[1].astype(jnp.int32), loops,
                           jnp.zeros((padn,), jnp.int32)])
    zer16 = jnp.zeros((N, 16), jnp.float32)
    zer128 = jnp.zeros((N, 128), jnp.float32)

    xcat = _mm(x, jnp.concatenate([Wl1, Wr1], axis=1))
    xl1, xr1 = xcat[:, :HEADS * HID], xcat[:, HEADS * HID:]
    ex1, den1 = _edge_ex(xl1, xr1, att1.reshape(-1), src, dst, zer16, HEADS)
    xlt1 = xl1.reshape(N, 8, 128).transpose(1, 0, 2).reshape(8 * N, 128)
    ch1 = _edge_aggr(xlt1, ex1, src, dst, zer128, 8)
    h1full = ch1.transpose(1, 0, 2).reshape(N, HEADS * HID)
    h1 = _norm(h1full, den1, b1, HEADS, relu=True)

    xcat2 = _mm(h1, jnp.concatenate([Wl2, Wr2], axis=1), bn=256)
    xl2, xr2 = xcat2[:, :HID], xcat2[:, HID:]
    ex2, den2 = _edge_ex(xl2, xr2, att2.reshape(-1), src, dst, zer16, 1)
    xlt2 = xl2.reshape(N, 2, 128).transpose(1, 0, 2).reshape(2 * N, 128)
    ch2 = _edge_aggr(xlt2, ex2, src, dst, zer128, 2)
    h2full = ch2.transpose(1, 0, 2).reshape(N, HID)
    h2 = _norm(h2full, den2, b2, 1, relu=False)

    out = _pool_readout(h2, Wro, bro)
    return out.reshape(1, TOK, IN)


# speedup vs baseline: 1.7662x; 1.0539x over previous
import functools
import jax
import jax.numpy as jnp
from jax import lax
from jax.experimental import pallas as pl
from jax.experimental.pallas import tpu as pltpu, tpu_sc as plsc

N = 10000
E = 160000
IN = 384
HID = 256
HEADS = 4
TOK = 4

NC, NS, L = 2, 16, 16
NW = NC * NS
E_REAL = E + N
EPAD = 172032
PER_TILE = EPAD // NW
NBATCH = PER_TILE // L
PER_TILE_SC = EPAD // NS
BB = 64
NBATCH_SC = PER_TILE_SC // BB
NPAIR = NBATCH_SC // 2
ROWS_PER_TILE = 624
ROWS_TAIL = N - NS * ROWS_PER_TILE


def _rows_copy(mk_src, mk_dst, sid):
    r0 = sid * ROWS_PER_TILE
    pltpu.sync_copy(mk_src(r0, ROWS_PER_TILE), mk_dst(r0, ROWS_PER_TILE))
    @pl.when(sid == NS - 1)
    def _():
        t0 = NS * ROWS_PER_TILE
        pltpu.sync_copy(mk_src(t0, ROWS_TAIL), mk_dst(t0, ROWS_TAIL))

_mesh = lambda: plsc.VectorSubcoreMesh(core_axis_name="c", subcore_axis_name="s",
                                       num_cores=NC, num_subcores=NS)


def _mm(x, w, bn=512):
    m, k = x.shape
    n = w.shape[1]
    bm = 400
    def body(xr, wr, outr):
        outr[...] = jnp.dot(xr[...], wr[...], preferred_element_type=jnp.float32)
    return pl.pallas_call(
        body,
        grid=(m // bm, n // bn),
        in_specs=[pl.BlockSpec((bm, k), lambda i, j: (i, 0)),
                  pl.BlockSpec((k, bn), lambda i, j: (0, j))],
        out_specs=pl.BlockSpec((bm, bn), lambda i, j: (i, j)),
        out_shape=jax.ShapeDtypeStruct((m, n), jnp.float32),
    )(x, w)


def _edge_ex(xl, xr, att_flat, src, dst, zer16, heads):
    hf = heads * HID

    def body(xl_hbm, xr_hbm, att_hbm, src_hbm, dst_hbm, zer_hbm,
             ex_hbm, den_hbm,
             att_v, srcv, dstv, dsti, xlr0, xrr0, xlr1, xrr1, exb, den_acc,
             sa0, sb0, sa1, sb1):
        cid = lax.axis_index("c")
        sid = lax.axis_index("s")
        wid = sid * NC + cid
        _rows_copy(lambda r, n: zer_hbm.at[pl.ds(r, n)],
                   lambda r, n: den_acc.at[pl.ds(r, n)], sid)
        pltpu.sync_copy(att_hbm, att_v)
        lane = lax.iota(jnp.int32, L)
        base_t = wid * PER_TILE
        pltpu.sync_copy(src_hbm.at[pl.ds(base_t, PER_TILE)], srcv)
        pltpu.sync_copy(dst_hbm.at[pl.ds(base_t, PER_TILE)], dstv)
        plsc.subcore_barrier()

        def zrow(i, carry):
            exb[i] = jnp.zeros((L,), jnp.float32)
            return carry
        lax.fori_loop(0, L, zrow, 0)

        def gathers(b, xlr, xrr, sa, sb):
            pltpu.async_copy(xl_hbm.at[srcv.at[pl.ds(b * L, L)]], xlr, sa)
            pltpu.async_copy(xr_hbm.at[dstv.at[pl.ds(b * L, L)]], xrr, sb)

        def gwait(b, xlr, xrr, sa, sb):
            pltpu.make_async_copy(
                xl_hbm.at[srcv.at[pl.ds(b * L, L)]], xlr, sa).wait()
            pltpu.make_async_copy(
                xr_hbm.at[dstv.at[pl.ds(b * L, L)]], xrr, sb).wait()

        def compute(b, xlr, xrr):
            base = base_t + b * L
            valid = (base + lane) < E_REAL
            for h in range(heads):
                def floop(f, accs):
                    a0, a1, a2, a3 = accs
                    outs = []
                    for j, acc in enumerate((a0, a1, a2, a3)):
                        col = jnp.full((L,), h * HID + f * 4 + j, jnp.int32)
                        a = plsc.load_gather(xlr, [lane, col])
                        bb = plsc.load_gather(xrr, [lane, col])
                        av = plsc.load_gather(att_v, [col])
                        t = a + bb
                        m = jnp.maximum(t, 0.2 * t)
                        outs.append(acc + m * av)
                    return tuple(outs)
                z = jnp.zeros((L,), jnp.float32)
                a0, a1, a2, a3 = lax.fori_loop(0, HID // 4, floop,
                                               (z, z, z, z), unroll=2)
                logits = (a0 + a1) + (a2 + a3)
                exv = jnp.where(valid, jnp.exp(logits), 0.0)
                plsc.store_scatter(exb, [lane, jnp.full((L,), h, jnp.int32)],
                                   exv)
            pltpu.sync_copy(dst_hbm.at[pl.ds(base, L)], dsti)
            pltpu.sync_copy(exb, den_acc.at[dsti], add=True)
            pltpu.sync_copy(exb, ex_hbm.at[pl.ds(base, L)])

        gathers(0, xlr0, xrr0, sa0, sb0)

        def pair(p, carry):
            b0 = 2 * p
            gwait(b0, xlr0, xrr0, sa0, sb0)
            gathers(b0 + 1, xlr1, xrr1, sa1, sb1)
            compute(b0, xlr0, xrr0)
            @pl.when(p < NBATCH // 2 - 1)
            def _():
                gathers(b0 + 2, xlr0, xrr0, sa0, sb0)
            gwait(b0 + 1, xlr1, xrr1, sa1, sb1)
            compute(b0 + 1, xlr1, xrr1)
            return carry
        lax.fori_loop(0, NBATCH // 2, pair, 0)
        plsc.subcore_barrier()
        _rows_copy(lambda r, n: den_acc.at[pl.ds(r, n)],
                   lambda r, n: den_hbm.at[cid, pl.ds(r, n)], sid)

    fn = pl.kernel(
        body,
        out_type=(jax.ShapeDtypeStruct((EPAD, 16), jnp.float32),
                  jax.ShapeDtypeStruct((2, N, 16), jnp.float32)),
        mesh=_mesh(),
        compiler_params=pltpu.CompilerParams(use_tc_tiling_on_sc=False, needs_layout_passes=False),
        scratch_types=[
            pltpu.VMEM((hf,), jnp.float32),
            pltpu.VMEM((PER_TILE,), jnp.int32),
            pltpu.VMEM((PER_TILE,), jnp.int32),
            pltpu.VMEM((L,), jnp.int32),
            pltpu.VMEM((L, hf), jnp.float32),
            pltpu.VMEM((L, hf), jnp.float32),
            pltpu.VMEM((L, hf), jnp.float32),
            pltpu.VMEM((L, hf), jnp.float32),
            pltpu.VMEM((L, 16), jnp.float32),
            pltpu.VMEM_SHARED((N, 16), jnp.float32),
            pltpu.SemaphoreType.DMA,
            pltpu.SemaphoreType.DMA,
            pltpu.SemaphoreType.DMA,
            pltpu.SemaphoreType.DMA,
        ],
    )
    return fn(xl, xr, att_flat, src, dst, zer16)


def _edge_aggr(xlt, ex, src, dst, zer128, nch):
    def body(xlt_hbm, ex_hbm, src_hbm, dst_hbm, zer_hbm,
             out_hbm,
             srcc, dsti0, dsti1, rows0, rows1, exb0, exb1, acc_sh,
             semg0, semg1, semx0, semx1, semd0, semd1):
        cid = lax.axis_index("c")
        sid = lax.axis_index("s")
        base_t = sid * PER_TILE_SC
        lane = lax.iota(jnp.int32, L)

        pltpu.sync_copy(src_hbm.at[pl.ds(base_t, PER_TILE_SC)], srcc)

        def shift(d):
            def sloop(j, carry):
                sl = pl.ds(j * L, L)
                srcc[sl] = srcc[sl] + d
                return carry
            lax.fori_loop(0, PER_TILE_SC // L, sloop, 0, unroll=8)

        shift(cid * N)
        for k in range(nch // 2):
            if k > 0:
                shift(2 * N)
            c = 2 * k + cid
            _rows_copy(lambda r, n: zer_hbm.at[pl.ds(r, n)],
                       lambda r, n: acc_sh.at[pl.ds(r, n)], sid)
            plsc.subcore_barrier()

            def scale_scatter(b, rows, exb, dsti):
                exvs = [plsc.load_gather(
                    exb, [lane + g * L, jnp.full((L,), k, jnp.int32)])
                    for g in range(BB // L)]

                def floop(f, carry2):
                    col = jnp.full((L,), f, jnp.int32)
                    for g in range(BB // L):
                        rl = lane + g * L
                        v = plsc.load_gather(rows, [rl, col])
                        plsc.store_scatter(rows, [rl, col], v * exvs[g])
                    return carry2
                lax.fori_loop(0, 128, floop, 0, unroll=2)
                pltpu.sync_copy(rows, acc_sh.at[dsti], add=True)

            def gather(b, rows, exb, dsti, sg, sx, sd):
                pltpu.async_copy(
                    xlt_hbm.at[srcc.at[pl.ds(b * BB, BB)]], rows, sg)
                pltpu.async_copy(
                    ex_hbm.at[pl.ds(base_t + b * BB, BB)], exb, sx)
                pltpu.async_copy(
                    dst_hbm.at[pl.ds(base_t + b * BB, BB)], dsti, sd)

            def gwait(b, rows, exb, dsti, sg, sx, sd):
                pltpu.make_async_copy(
                    xlt_hbm.at[srcc.at[pl.ds(b * BB, BB)]], rows, sg).wait()
                pltpu.make_async_copy(
                    ex_hbm.at[pl.ds(base_t + b * BB, BB)], exb, sx).wait()
                pltpu.make_async_copy(
                    dst_hbm.at[pl.ds(base_t + b * BB, BB)], dsti, sd).wait()

            gather(0, rows0, exb0, dsti0, semg0, semx0, semd0)

            def pair(p, carry):
                b0 = 2 * p
                gwait(b0, rows0, exb0, dsti0, semg0, semx0, semd0)
                gather(b0 + 1, rows1, exb1, dsti1, semg1, semx1, semd1)
                scale_scatter(b0, rows0, exb0, dsti0)
                @pl.when(p < NPAIR - 1)
                def _():
                    gather(b0 + 2, rows0, exb0, dsti0, semg0, semx0, semd0)
                gwait(b0 + 1, rows1, exb1, dsti1, semg1, semx1, semd1)
                scale_scatter(b0 + 1, rows1, exb1, dsti1)
                return carry
            lax.fori_loop(0, NPAIR, pair, 0)
            plsc.subcore_barrier()
            _rows_copy(lambda r, n: acc_sh.at[pl.ds(r, n)],
                       lambda r, n: out_hbm.at[c, pl.ds(r, n)], sid)
            plsc.subcore_barrier()

    fn = pl.kernel(
        body,
        out_type=jax.ShapeDtypeStruct((nch, N, 128), jnp.float32),
        mesh=_mesh(),
        compiler_params=pltpu.CompilerParams(use_tc_tiling_on_sc=False, needs_layout_passes=False),
        scratch_types=[
            pltpu.VMEM((PER_TILE_SC,), jnp.int32),
            pltpu.VMEM((BB,), jnp.int32),
            pltpu.VMEM((BB,), jnp.int32),
            pltpu.VMEM((BB, 128), jnp.float32),
            pltpu.VMEM((BB, 128), jnp.float32),
            pltpu.VMEM((BB, 16), jnp.float32),
            pltpu.VMEM((BB, 16), jnp.float32),
            pltpu.VMEM_SHARED((N, 128), jnp.float32),
            pltpu.SemaphoreType.DMA,
            pltpu.SemaphoreType.DMA,
            pltpu.SemaphoreType.DMA,
            pltpu.SemaphoreType.DMA,
            pltpu.SemaphoreType.DMA,
            pltpu.SemaphoreType.DMA,
        ],
    )
    return fn(xlt, ex, src, dst, zer128)


def _norm(chunks, den, bias, heads, relu):
    hf = heads * HID
    bm = 400
    def body(xr, dr, br, outr):
        x = xr[...]
        d = dr[...][0] + dr[...][1]
        b = br[...]
        cols = []
        for h in range(heads):
            seg = x[:, h * HID:(h + 1) * HID]
            dh = d[:, h:h + 1] + 1e-16
            cols.append(seg / dh)
        o = jnp.concatenate(cols, axis=1) + b
        if relu:
            o = jnp.maximum(o, 0.0)
        outr[...] = o
    return pl.pallas_call(
        body,
        grid=(N // bm,),
        in_specs=[pl.BlockSpec((bm, hf), lambda i: (i, 0)),
                  pl.BlockSpec((2, bm, 16), lambda i: (0, i, 0)),
                  pl.BlockSpec((1, hf), lambda i: (0, 0))],
        out_specs=pl.BlockSpec((bm, hf), lambda i: (i, 0)),
        out_shape=jax.ShapeDtypeStruct((N, hf), jnp.float32),
    )(chunks, den, bias.reshape(1, hf))


def _pool_readout(h, wro, bro):
    bm = 400
    nsteps = N // bm
    def body(hr, wr, br, outr, acc):
        i = pl.program_id(0)
        @pl.when(i == 0)
        def _():
            acc[...] = jnp.zeros_like(acc)
        acc[0:1, :] += jnp.sum(hr[...], axis=0, keepdims=True)
        @pl.when(i == nsteps - 1)
        def _():
            pooled = acc[0:1, :] / float(N)
            outr[...] = jnp.dot(pooled, wr[...],
                                preferred_element_type=jnp.float32) + br[...]
    return pl.pallas_call(
        body,
        grid=(nsteps,),
        in_specs=[pl.BlockSpec((bm, HID), lambda i: (i, 0)),
                  pl.BlockSpec((HID, TOK * IN), lambda i: (0, 0)),
                  pl.BlockSpec((1, TOK * IN), lambda i: (0, 0))],
        out_specs=pl.BlockSpec((1, TOK * IN), lambda i: (0, 0)),
        out_shape=jax.ShapeDtypeStruct((1, TOK * IN), jnp.float32),
        scratch_shapes=[pltpu.VMEM((8, HID), jnp.float32)],
    )(h, wro, bro.reshape(1, TOK * IN))


def kernel(x, edge_index, batch, Wl1, Wr1, att1, b1, Wl2, Wr2, att2, b2, Wro, bro):
    loops = jnp.arange(N, dtype=jnp.int32)
    padn = EPAD - E_REAL
    src = jnp.concatenate([edge_index[0].astype(jnp.int32), loops,
                           jnp.zeros((padn,), jnp.int32)])
    dst = jnp.concatenate([edge_index[1].astype(jnp.int32), loops,
                           jnp.zeros((padn,), jnp.int32)])
    zer16 = jnp.zeros((N, 16), jnp.float32)
    zer128 = jnp.zeros((N, 128), jnp.float32)

    xcat = _mm(x, jnp.concatenate([Wl1, Wr1], axis=1))
    xl1, xr1 = xcat[:, :HEADS * HID], xcat[:, HEADS * HID:]
    ex1, den1 = _edge_ex(xl1, xr1, att1.reshape(-1), src, dst, zer16, HEADS)
    xlt1 = xl1.reshape(N, 8, 128).transpose(1, 0, 2).reshape(8 * N, 128)
    ch1 = _edge_aggr(xlt1, ex1, src, dst, zer128, 8)
    h1full = ch1.transpose(1, 0, 2).reshape(N, HEADS * HID)
    h1 = _norm(h1full, den1, b1, HEADS, relu=True)

    xcat2 = _mm(h1, jnp.concatenate([Wl2, Wr2], axis=1), bn=256)
    xl2, xr2 = xcat2[:, :HID], xcat2[:, HID:]
    ex2, den2 = _edge_ex(xl2, xr2, att2.reshape(-1), src, dst, zer16, 1)
    xlt2 = xl2.reshape(N, 2, 128).transpose(1, 0, 2).reshape(2 * N, 128)
    ch2 = _edge_aggr(xlt2, ex2, src, dst, zer128, 2)
    h2full = ch2.transpose(1, 0, 2).reshape(N, HID)
    h2 = _norm(h2full, den2, b2, 1, relu=False)

    out = _pool_readout(h2, Wro, bro)
    return out.reshape(1, TOK, IN)


# speedup vs baseline: 1.7702x; 1.0023x over previous
import functools
import jax
import jax.numpy as jnp
from jax import lax
from jax.experimental import pallas as pl
from jax.experimental.pallas import tpu as pltpu, tpu_sc as plsc

N = 10000
E = 160000
IN = 384
HID = 256
HEADS = 4
TOK = 4

NC, NS, L = 2, 16, 16
NW = NC * NS
E_REAL = E + N
EPAD = 172032
PER_TILE = EPAD // NW
NBATCH = PER_TILE // L
PER_TILE_SC = EPAD // NS
BB = 128
NBATCH_SC = PER_TILE_SC // BB
NPAIR = NBATCH_SC // 2
ROWS_PER_TILE = 624
ROWS_TAIL = N - NS * ROWS_PER_TILE


def _rows_copy(mk_src, mk_dst, sid):
    r0 = sid * ROWS_PER_TILE
    pltpu.sync_copy(mk_src(r0, ROWS_PER_TILE), mk_dst(r0, ROWS_PER_TILE))
    @pl.when(sid == NS - 1)
    def _():
        t0 = NS * ROWS_PER_TILE
        pltpu.sync_copy(mk_src(t0, ROWS_TAIL), mk_dst(t0, ROWS_TAIL))

_mesh = lambda: plsc.VectorSubcoreMesh(core_axis_name="c", subcore_axis_name="s",
                                       num_cores=NC, num_subcores=NS)


def _mm(x, w, bn=512):
    m, k = x.shape
    n = w.shape[1]
    bm = 400
    def body(xr, wr, outr):
        outr[...] = jnp.dot(xr[...], wr[...], preferred_element_type=jnp.float32)
    return pl.pallas_call(
        body,
        grid=(m // bm, n // bn),
        in_specs=[pl.BlockSpec((bm, k), lambda i, j: (i, 0)),
                  pl.BlockSpec((k, bn), lambda i, j: (0, j))],
        out_specs=pl.BlockSpec((bm, bn), lambda i, j: (i, j)),
        out_shape=jax.ShapeDtypeStruct((m, n), jnp.float32),
    )(x, w)


def _edge_ex(xl, xr, att_flat, src, dst, zer16, heads):
    hf = heads * HID

    def body(xl_hbm, xr_hbm, att_hbm, src_hbm, dst_hbm, zer_hbm,
             ex_hbm, den_hbm,
             att_v, srcv, dstv, dsti, xlr0, xrr0, xlr1, xrr1, exb, den_acc,
             sa0, sb0, sa1, sb1):
        cid = lax.axis_index("c")
        sid = lax.axis_index("s")
        wid = sid * NC + cid
        _rows_copy(lambda r, n: zer_hbm.at[pl.ds(r, n)],
                   lambda r, n: den_acc.at[pl.ds(r, n)], sid)
        pltpu.sync_copy(att_hbm, att_v)
        lane = lax.iota(jnp.int32, L)
        base_t = wid * PER_TILE
        pltpu.sync_copy(src_hbm.at[pl.ds(base_t, PER_TILE)], srcv)
        pltpu.sync_copy(dst_hbm.at[pl.ds(base_t, PER_TILE)], dstv)
        plsc.subcore_barrier()

        def zrow(i, carry):
            exb[i] = jnp.zeros((L,), jnp.float32)
            return carry
        lax.fori_loop(0, L, zrow, 0)

        def gathers(b, xlr, xrr, sa, sb):
            pltpu.async_copy(xl_hbm.at[srcv.at[pl.ds(b * L, L)]], xlr, sa)
            pltpu.async_copy(xr_hbm.at[dstv.at[pl.ds(b * L, L)]], xrr, sb)

        def gwait(b, xlr, xrr, sa, sb):
            pltpu.make_async_copy(
                xl_hbm.at[srcv.at[pl.ds(b * L, L)]], xlr, sa).wait()
            pltpu.make_async_copy(
                xr_hbm.at[dstv.at[pl.ds(b * L, L)]], xrr, sb).wait()

        def compute(b, xlr, xrr):
            base = base_t + b * L
            valid = (base + lane) < E_REAL
            for h in range(heads):
                def floop(f, accs):
                    a0, a1, a2, a3 = accs
                    outs = []
                    for j, acc in enumerate((a0, a1, a2, a3)):
                        col = jnp.full((L,), h * HID + f * 4 + j, jnp.int32)
                        a = plsc.load_gather(xlr, [lane, col])
                        bb = plsc.load_gather(xrr, [lane, col])
                        av = plsc.load_gather(att_v, [col])
                        t = a + bb
                        m = jnp.maximum(t, 0.2 * t)
                        outs.append(acc + m * av)
                    return tuple(outs)
                z = jnp.zeros((L,), jnp.float32)
                a0, a1, a2, a3 = lax.fori_loop(0, HID // 4, floop,
                                               (z, z, z, z), unroll=2)
                logits = (a0 + a1) + (a2 + a3)
                exv = jnp.where(valid, jnp.exp(logits), 0.0)
                plsc.store_scatter(exb, [lane, jnp.full((L,), h, jnp.int32)],
                                   exv)
            pltpu.sync_copy(dst_hbm.at[pl.ds(base, L)], dsti)
            pltpu.sync_copy(exb, den_acc.at[dsti], add=True)
            pltpu.sync_copy(exb, ex_hbm.at[pl.ds(base, L)])

        gathers(0, xlr0, xrr0, sa0, sb0)

        def pair(p, carry):
            b0 = 2 * p
            gwait(b0, xlr0, xrr0, sa0, sb0)
            gathers(b0 + 1, xlr1, xrr1, sa1, sb1)
            compute(b0, xlr0, xrr0)
            @pl.when(p < NBATCH // 2 - 1)
            def _():
                gathers(b0 + 2, xlr0, xrr0, sa0, sb0)
            gwait(b0 + 1, xlr1, xrr1, sa1, sb1)
            compute(b0 + 1, xlr1, xrr1)
            return carry
        lax.fori_loop(0, NBATCH // 2, pair, 0)
        plsc.subcore_barrier()
        _rows_copy(lambda r, n: den_acc.at[pl.ds(r, n)],
                   lambda r, n: den_hbm.at[cid, pl.ds(r, n)], sid)

    fn = pl.kernel(
        body,
        out_type=(jax.ShapeDtypeStruct((EPAD, 16), jnp.float32),
                  jax.ShapeDtypeStruct((2, N, 16), jnp.float32)),
        mesh=_mesh(),
        compiler_params=pltpu.CompilerParams(use_tc_tiling_on_sc=False, needs_layout_passes=False),
        scratch_types=[
            pltpu.VMEM((hf,), jnp.float32),
            pltpu.VMEM((PER_TILE,), jnp.int32),
            pltpu.VMEM((PER_TILE,), jnp.int32),
            pltpu.VMEM((L,), jnp.int32),
            pltpu.VMEM((L, hf), jnp.float32),
            pltpu.VMEM((L, hf), jnp.float32),
            pltpu.VMEM((L, hf), jnp.float32),
            pltpu.VMEM((L, hf), jnp.float32),
            pltpu.VMEM((L, 16), jnp.float32),
            pltpu.VMEM_SHARED((N, 16), jnp.float32),
            pltpu.SemaphoreType.DMA,
            pltpu.SemaphoreType.DMA,
            pltpu.SemaphoreType.DMA,
            pltpu.SemaphoreType.DMA,
        ],
    )
    return fn(xl, xr, att_flat, src, dst, zer16)


def _edge_aggr(xlt, ex, src, dst, zer128, nch):
    def body(xlt_hbm, ex_hbm, src_hbm, dst_hbm, zer_hbm,
             out_hbm,
             srcc, dsti0, dsti1, rows0, rows1, exb0, exb1, acc_sh,
             semg0, semg1, semx0, semx1, semd0, semd1):
        cid = lax.axis_index("c")
        sid = lax.axis_index("s")
        base_t = sid * PER_TILE_SC
        lane = lax.iota(jnp.int32, L)

        pltpu.sync_copy(src_hbm.at[pl.ds(base_t, PER_TILE_SC)], srcc)

        def shift(d):
            def sloop(j, carry):
                sl = pl.ds(j * L, L)
                srcc[sl] = srcc[sl] + d
                return carry
            lax.fori_loop(0, PER_TILE_SC // L, sloop, 0, unroll=8)

        shift(cid * N)
        for k in range(nch // 2):
            if k > 0:
                shift(2 * N)
            c = 2 * k + cid
            _rows_copy(lambda r, n: zer_hbm.at[pl.ds(r, n)],
                       lambda r, n: acc_sh.at[pl.ds(r, n)], sid)
            plsc.subcore_barrier()

            def scale_scatter(b, rows, exb, dsti):
                exvs = [plsc.load_gather(
                    exb, [lane + g * L, jnp.full((L,), k, jnp.int32)])
                    for g in range(BB // L)]

                def floop(f, carry2):
                    col = jnp.full((L,), f, jnp.int32)
                    for g in range(BB // L):
                        rl = lane + g * L
                        v = plsc.load_gather(rows, [rl, col])
                        plsc.store_scatter(rows, [rl, col], v * exvs[g])
                    return carry2
                lax.fori_loop(0, 128, floop, 0, unroll=2)
                pltpu.sync_copy(rows, acc_sh.at[dsti], add=True)

            def gather(b, rows, exb, dsti, sg, sx, sd):
                pltpu.async_copy(
                    xlt_hbm.at[srcc.at[pl.ds(b * BB, BB)]], rows, sg)
                pltpu.async_copy(
                    ex_hbm.at[pl.ds(base_t + b * BB, BB)], exb, sx)
                pltpu.async_copy(
                    dst_hbm.at[pl.ds(base_t + b * BB, BB)], dsti, sd)

            def gwait(b, rows, exb, dsti, sg, sx, sd):
                pltpu.make_async_copy(
                    xlt_hbm.at[srcc.at[pl.ds(b * BB, BB)]], rows, sg).wait()
                pltpu.make_async_copy(
                    ex_hbm.at[pl.ds(base_t + b * BB, BB)], exb, sx).wait()
                pltpu.make_async_copy(
                    dst_hbm.at[pl.ds(base_t + b * BB, BB)], dsti, sd).wait()

            gather(0, rows0, exb0, dsti0, semg0, semx0, semd0)

            def pair(p, carry):
                b0 = 2 * p
                gwait(b0, rows0, exb0, dsti0, semg0, semx0, semd0)
                gather(b0 + 1, rows1, exb1, dsti1, semg1, semx1, semd1)
                scale_scatter(b0, rows0, exb0, dsti0)
                @pl.when(p < NPAIR - 1)
                def _():
                    gather(b0 + 2, rows0, exb0, dsti0, semg0, semx0, semd0)
                gwait(b0 + 1, rows1, exb1, dsti1, semg1, semx1, semd1)
                scale_scatter(b0 + 1, rows1, exb1, dsti1)
                return carry
            lax.fori_loop(0, NPAIR, pair, 0)
            plsc.subcore_barrier()
            _rows_copy(lambda r, n: acc_sh.at[pl.ds(r, n)],
                       lambda r, n: out_hbm.at[c, pl.ds(r, n)], sid)
            plsc.subcore_barrier()

    fn = pl.kernel(
        body,
        out_type=jax.ShapeDtypeStruct((nch, N, 128), jnp.float32),
        mesh=_mesh(),
        compiler_params=pltpu.CompilerParams(use_tc_tiling_on_sc=False, needs_layout_passes=False),
        scratch_types=[
            pltpu.VMEM((PER_TILE_SC,), jnp.int32),
            pltpu.VMEM((BB,), jnp.int32),
            pltpu.VMEM((BB,), jnp.int32),
            pltpu.VMEM((BB, 128), jnp.float32),
            pltpu.VMEM((BB, 128), jnp.float32),
            pltpu.VMEM((BB, 16), jnp.float32),
            pltpu.VMEM((BB, 16), jnp.float32),
            pltpu.VMEM_SHARED((N, 128), jnp.float32),
            pltpu.SemaphoreType.DMA,
            pltpu.SemaphoreType.DMA,
            pltpu.SemaphoreType.DMA,
            pltpu.SemaphoreType.DMA,
            pltpu.SemaphoreType.DMA,
            pltpu.SemaphoreType.DMA,
        ],
    )
    return fn(xlt, ex, src, dst, zer128)


def _norm(chunks, den, bias, heads, relu):
    hf = heads * HID
    bm = 400
    def body(xr, dr, br, outr):
        x = xr[...]
        d = dr[...][0] + dr[...][1]
        b = br[...]
        cols = []
        for h in range(heads):
            seg = x[:, h * HID:(h + 1) * HID]
            dh = d[:, h:h + 1] + 1e-16
            cols.append(seg / dh)
        o = jnp.concatenate(cols, axis=1) + b
        if relu:
            o = jnp.maximum(o, 0.0)
        outr[...] = o
    return pl.pallas_call(
        body,
        grid=(N // bm,),
        in_specs=[pl.BlockSpec((bm, hf), lambda i: (i, 0)),
                  pl.BlockSpec((2, bm, 16), lambda i: (0, i, 0)),
                  pl.BlockSpec((1, hf), lambda i: (0, 0))],
        out_specs=pl.BlockSpec((bm, hf), lambda i: (i, 0)),
        out_shape=jax.ShapeDtypeStruct((N, hf), jnp.float32),
    )(chunks, den, bias.reshape(1, hf))


def _pool_readout(h, wro, bro):
    bm = 400
    nsteps = N // bm
    def body(hr, wr, br, outr, acc):
        i = pl.program_id(0)
        @pl.when(i == 0)
        def _():
            acc[...] = jnp.zeros_like(acc)
        acc[0:1, :] += jnp.sum(hr[...], axis=0, keepdims=True)
        @pl.when(i == nsteps - 1)
        def _():
            pooled = acc[0:1, :] / float(N)
            outr[...] = jnp.dot(pooled, wr[...],
                                preferred_element_type=jnp.float32) + br[...]
    return pl.pallas_call(
        body,
        grid=(nsteps,),
        in_specs=[pl.BlockSpec((bm, HID), lambda i: (i, 0)),
                  pl.BlockSpec((HID, TOK * IN), lambda i: (0, 0)),
                  pl.BlockSpec((1, TOK * IN), lambda i: (0, 0))],
        out_specs=pl.BlockSpec((1, TOK * IN), lambda i: (0, 0)),
        out_shape=jax.ShapeDtypeStruct((1, TOK * IN), jnp.float32),
        scratch_shapes=[pltpu.VMEM((8, HID), jnp.float32)],
    )(h, wro, bro.reshape(1, TOK * IN))


def kernel(x, edge_index, batch, Wl1, Wr1, att1, b1, Wl2, Wr2, att2, b2, Wro, bro):
    loops = jnp.arange(N, dtype=jnp.int32)
    padn = EPAD - E_REAL
    src = jnp.concatenate([edge_index[0].astype(jnp.int32), loops,
                           jnp.zeros((padn,), jnp.int32)])
    dst = jnp.concatenate([edge_index[1].astype(jnp.int32), loops,
                           jnp.zeros((padn,), jnp.int32)])
    zer16 = jnp.zeros((N, 16), jnp.float32)
    zer128 = jnp.zeros((N, 128), jnp.float32)

    xcat = _mm(x, jnp.concatenate([Wl1, Wr1], axis=1))
    xl1, xr1 = xcat[:, :HEADS * HID], xcat[:, HEADS * HID:]
    ex1, den1 = _edge_ex(xl1, xr1, att1.reshape(-1), src, dst, zer16, HEADS)
    xlt1 = xl1.reshape(N, 8, 128).transpose(1, 0, 2).reshape(8 * N, 128)
    ch1 = _edge_aggr(xlt1, ex1, src, dst, zer128, 8)
    h1full = ch1.transpose(1, 0, 2).reshape(N, HEADS * HID)
    h1 = _norm(h1full, den1, b1, HEADS, relu=True)

    xcat2 = _mm(h1, jnp.concatenate([Wl2, Wr2], axis=1), bn=256)
    xl2, xr2 = xcat2[:, :HID], xcat2[:, HID:]
    ex2, den2 = _edge_ex(xl2, xr2, att2.reshape(-1), src, dst, zer16, 1)
    xlt2 = xl2.reshape(N, 2, 128).transpose(1, 0, 2).reshape(2 * N, 128)
    ch2 = _edge_aggr(xlt2, ex2, src, dst, zer128, 2)
    h2full = ch2.transpose(1, 0, 2).reshape(N, HID)
    h2 = _norm(h2full, den2, b2, 1, relu=False)

    out = _pool_readout(h2, Wro, bro)
    return out.reshape(1, TOK, IN)
